# baseline (speedup 1.0000x reference)
import functools

import jax
import jax.numpy as jnp
from jax import lax
from jax.experimental import pallas as pl
from jax.experimental.pallas import tpu as pltpu
from jax.experimental.pallas import tpu_sc as plsc

B = 16384
F = 26
V = 100000
D = 16
H = 200

NC = 2
NS = 16
NW = NC * NS

G = 128
NEG = B // 16
EGPW = NEG // NW
NWG = (B * 32) // G
WGPW = NWG // NW


def _sc_gather(i0, i1, i2, i3, iw, tab2d, w_flat):
    mesh = plsc.VectorSubcoreMesh(core_axis_name="c", subcore_axis_name="s")

    @functools.partial(
        pl.kernel,
        mesh=mesh,
        compiler_params=pltpu.CompilerParams(use_tc_tiling_on_sc=False),
        out_type=[
            jax.ShapeDtypeStruct((NEG, G, D), jnp.float32),
            jax.ShapeDtypeStruct((NEG, G, D), jnp.float32),
            jax.ShapeDtypeStruct((NEG, G, D), jnp.float32),
            jax.ShapeDtypeStruct((NEG, G, D), jnp.float32),
            jax.ShapeDtypeStruct((NWG, G), jnp.float32),
        ],
        scratch_types=[
            pltpu.VMEM((4 * EGPW, G), jnp.int32),
            pltpu.VMEM((WGPW, G), jnp.int32),
            pltpu.VMEM((G, D), jnp.float32),
            pltpu.VMEM((G, D), jnp.float32),
            pltpu.VMEM((G,), jnp.float32),
            pltpu.VMEM((G,), jnp.float32),
            pltpu.VMEM((G,), jnp.float32),
            pltpu.VMEM((G,), jnp.float32),
            pltpu.SemaphoreType.DMA,
            pltpu.SemaphoreType.DMA,
            pltpu.SemaphoreType.DMA,
            pltpu.SemaphoreType.DMA,
        ],
    )
    def gather_kernel(i0_h, i1_h, i2_h, i3_h, iw_h, tab_h, w_h,
                      e0_h, e1_h, e2_h, e3_h, wv_h,
                      idxe_v, idxw_v, bufa, bufb, wba, wbb, wbc, wbd,
                      sema, semb, semc, semd):
        wid = lax.axis_index("s") * NC + lax.axis_index("c")
        eb = wid * EGPW
        wb = wid * WGPW
        for q, iq_h in enumerate((i0_h, i1_h, i2_h, i3_h)):
            pltpu.sync_copy(iq_h.at[pl.ds(eb, EGPW)],
                            idxe_v.at[pl.ds(q * EGPW, EGPW)])
        pltpu.sync_copy(iw_h.at[pl.ds(wb, WGPW)], idxw_v)

        for q, eq_h in enumerate((e0_h, e1_h, e2_h, e3_h)):
            def ebody(j, carry, q=q, eq_h=eq_h):
                k = 2 * j
                cpa = pltpu.async_copy(
                    tab_h.at[idxe_v.at[q * EGPW + k]], bufa, sema)
                cpb = pltpu.async_copy(
                    tab_h.at[idxe_v.at[q * EGPW + k + 1]], bufb, semb)
                cpa.wait()
                pltpu.sync_copy(bufa, eq_h.at[eb + k])
                cpb.wait()
                pltpu.sync_copy(bufb, eq_h.at[eb + k + 1])
                return carry
            lax.fori_loop(0, EGPW // 2, ebody, 0)

        def wbody(j, carry):
            g = 4 * j
            cps = []
            for t, (wbuf, sem) in enumerate(
                    ((wba, sema), (wbb, semb), (wbc, semc), (wbd, semd))):
                cps.append(pltpu.async_copy(
                    w_h.at[idxw_v.at[g + t]], wbuf, sem))
            for t, (wbuf, _) in enumerate(
                    ((wba, sema), (wbb, semb), (wbc, semc), (wbd, semd))):
                cps[t].wait()
                pltpu.sync_copy(wbuf, wv_h.at[wb + g + t])
            return carry
        lax.fori_loop(0, WGPW // 4, wbody, 0)

    return gather_kernel(i0, i1, i2, i3, iw, tab2d, w_flat)


KV = 11
CW = KV * 128
CPF = 781 // KV
NCHK = F * CPF
CPW = 58
CROWS = CW // 8
VMAIN = 781 * 128
MROWS = F * (VMAIN // 8)
TROWS = MROWS + F * 8


def _sc_detile(tab_fd, tail):
    mesh = plsc.VectorSubcoreMesh(core_axis_name="c", subcore_axis_name="s")

    @functools.partial(
        pl.kernel,
        mesh=mesh,
        compiler_params=pltpu.CompilerParams(use_tc_tiling_on_sc=True,
                                             needs_layout_passes=False),
        out_type=[jax.ShapeDtypeStruct((TROWS, 128), jnp.float32)],
        scratch_types=[
            pltpu.VMEM((D, CW + 1), jnp.float32),
            pltpu.VMEM((D, CW + 1), jnp.float32),
            pltpu.VMEM((CROWS, 128), jnp.float32),
            pltpu.VMEM((CROWS, 128), jnp.float32),
            pltpu.SemaphoreType.DMA,
            pltpu.SemaphoreType.DMA,
            pltpu.SemaphoreType.DMA,
            pltpu.SemaphoreType.DMA,
        ],
    )
    def detile_kernel(tab_h, tail_h, out_h, bufa, bufb, ova, ovb,
                      sia, sib, soa, sob):
        wid = lax.axis_index("s") * NC + lax.axis_index("c")
        iota16 = lax.broadcasted_iota(jnp.int32, (D,), 0)

        def chunk_c(k):
            return k * NW + wid

        def fire_in(c, buf, sem):
            f = c // CPF
            vq = c - f * CPF
            pltpu.async_copy(
                tab_h.at[pl.ds(D * f, D), pl.ds(CW * vq, CW)],
                buf.at[pl.ds(0, D), pl.ds(0, CW)], sem)

        def wait_in(buf, sem):
            pltpu.make_async_copy(
                tab_h.at[pl.ds(0, D), pl.ds(0, CW)],
                buf.at[pl.ds(0, D), pl.ds(0, CW)], sem).wait()

        def shuffle(buf, ov):
            def jbody(j, carry):
                j8 = 8 * j
                for tt in range(KV):
                    for m in range(8):
                        colv = jnp.full((D,), 128 * tt + m, jnp.int32) + j8
                        val = plsc.load_gather(buf, [iota16, colv])
                        ov[D * tt + j, pl.ds(D * m, D)] = val
                return carry
            lax.fori_loop(0, 16, jbody, 0)

        def fire_out(c, ov, sem):
            pltpu.async_copy(ov, out_h.at[pl.ds(CROWS * c, CROWS)], sem)

        def wait_out(ov, sem):
            pltpu.make_async_copy(
                out_h.at[pl.ds(0, CROWS)], ov, sem).wait()

        @pl.when(chunk_c(0) < NCHK)
        def _():
            fire_in(chunk_c(0), bufa, sia)

        @pl.when(chunk_c(1) < NCHK)
        def _():
            fire_in(chunk_c(1), bufb, sib)

        def body(i, carry):
            k0 = 2 * i
            for par, (buf, ov, si, so) in enumerate(
                    ((bufa, ova, sia, soa), (bufb, ovb, sib, sob))):
                k = k0 + par
                c = chunk_c(k)

                @pl.when(c < NCHK)
                def _(buf=buf, ov=ov, si=si, so=so, k=k, c=c):
                    wait_in(buf, si)

                    @pl.when(k >= 2)
                    def _(ov=ov, so=so):
                        wait_out(ov, so)
                    shuffle(buf, ov)

                    @pl.when(chunk_c(k + 2) < NCHK)
                    def _(buf=buf, si=si, k=k):
                        fire_in(chunk_c(k + 2), buf, si)
                    fire_out(c, ov, so)
            return carry

        lax.fori_loop(0, CPW // 2, body, 0)

        @pl.when(chunk_c(CPW - 2) < NCHK)
        def _():
            wait_out(ova, soa)

        @pl.when(chunk_c(CPW - 1) < NCHK)
        def _():
            wait_out(ovb, sob)

        @pl.when(wid < F)
        def _():
            pltpu.sync_copy(tail_h.at[wid],
                            out_h.at[pl.ds(MROWS + 8 * wid, 8)])

    return detile_kernel(tab_fd, tail)[0]


BLK = 512


def _tc_head(e0, e1, e2, e3, wv, W1x, b1, W2, b2, W3, b3, Wd, bd):
    def body(e0_ref, e1_ref, e2_ref, e3_ref, wv_ref, W1x_ref, b1_ref,
             W2_ref, b2_ref, W3_ref, b3_ref, Wd_ref, bd_ref, out_ref):
        E0, E1, E2, E3 = e0_ref[...], e1_ref[...], e2_ref[...], e3_ref[...]
        m3 = (lax.broadcasted_iota(jnp.int32, (1, G), 1) < 2 * D).astype(
            jnp.float32)
        E3m = E3 * m3
        rj = lax.broadcasted_iota(jnp.int32, (G, D), 0)
        cd = lax.broadcasted_iota(jnp.int32, (G, D), 1)
        S = jnp.where((rj % D) == cd, 1.0, 0.0).astype(jnp.float32)
        Esum = E0 + E1 + E2 + E3m
        sum_d = jnp.dot(Esum, S, preferred_element_type=jnp.float32)
        sos = jnp.sum(sum_d * sum_d, axis=1, keepdims=True)
        ssq = jnp.sum(E0 * E0 + E1 * E1 + E2 * E2 + E3m * E3m,
                      axis=1, keepdims=True)
        second = 0.5 * (sos - ssq)
        mw = (lax.broadcasted_iota(jnp.int32, (1, 32), 1) < F).astype(
            jnp.float32)
        first = jnp.sum(wv_ref[...] * mw, axis=1, keepdims=True)
        W1x = W1x_ref[...]
        h = (jnp.dot(E0, W1x[0:128], preferred_element_type=jnp.float32)
             + jnp.dot(E1, W1x[128:256], preferred_element_type=jnp.float32)
             + jnp.dot(E2, W1x[256:384], preferred_element_type=jnp.float32)
             + jnp.dot(E3, W1x[384:512], preferred_element_type=jnp.float32))
        h = jnp.maximum(h + b1_ref[...], 0.0)
        h = jnp.maximum(
            jnp.dot(h, W2_ref[...], preferred_element_type=jnp.float32)
            + b2_ref[...], 0.0)
        h = jnp.maximum(
            jnp.dot(h, W3_ref[...], preferred_element_type=jnp.float32)
            + b3_ref[...], 0.0)
        deep = jnp.dot(h, Wd_ref[...], preferred_element_type=jnp.float32) \
            + bd_ref[...]
        out_ref[...] = jax.nn.sigmoid(first + second + deep)

    return pl.pallas_call(
        body,
        grid=(B // BLK,),
        in_specs=[
            pl.BlockSpec((BLK, G), lambda i: (i, 0)),
            pl.BlockSpec((BLK, G), lambda i: (i, 0)),
            pl.BlockSpec((BLK, G), lambda i: (i, 0)),
            pl.BlockSpec((BLK, G), lambda i: (i, 0)),
            pl.BlockSpec((BLK, 32), lambda i: (i, 0)),
            pl.BlockSpec((512, H), lambda i: (0, 0)),
            pl.BlockSpec((1, H), lambda i: (0, 0)),
            pl.BlockSpec((H, H), lambda i: (0, 0)),
            pl.BlockSpec((1, H), lambda i: (0, 0)),
            pl.BlockSpec((H, H), lambda i: (0, 0)),
            pl.BlockSpec((1, H), lambda i: (0, 0)),
            pl.BlockSpec((H, 1), lambda i: (0, 0)),
            pl.BlockSpec((1, 1), lambda i: (0, 0)),
        ],
        out_specs=pl.BlockSpec((BLK, 1), lambda i: (i, 0)),
        out_shape=jax.ShapeDtypeStruct((B, 1), jnp.float32),
    )(e0, e1, e2, e3, wv, W1x, b1.reshape(1, H), W2, b2.reshape(1, H),
      W3, b3.reshape(1, H), Wd, bd.reshape(1, 1))


def kernel(indices, tables, w_fm, W1, b1, W2, b2, W3, b3, Wd, bd):
    frow = jnp.arange(F, dtype=jnp.int32)[None, :]
    flat = indices + frow * V
    w32 = jnp.concatenate([flat, flat[:, :32 - F]], axis=1)
    iw = w32.reshape(NWG, G)
    v = indices
    vq = v // CW
    vloc = v - vq * CW
    main_row = 8 * (CROWS * (frow * CPF + vq) + vloc // 8) + (v % 8)
    tail_row = 8 * (MROWS + 8 * frow + (v - (V - 64)) // 8) + (v % 8)
    srow = jnp.where(v < VMAIN, main_row, tail_row)
    s32 = jnp.concatenate([srow, srow[:, :32 - F]], axis=1)
    iqs = [s32[:, 8 * q:8 * q + 8].reshape(NEG, G) for q in range(4)]
    tab_fd = tables.transpose(0, 2, 1).reshape(F * D, V)
    tail = tables[:, V - 64:, :].reshape(F, 8, 8 * D)
    tab2d = _sc_detile(tab_fd, tail).reshape(TROWS * 8, D)
    w_flat = w_fm.reshape(F * V)
    E0, E1, E2, E3, WV = _sc_gather(iqs[0], iqs[1], iqs[2], iqs[3], iw,
                                    tab2d, w_flat)
    e0 = E0.reshape(B, G)
    e1 = E1.reshape(B, G)
    e2 = E2.reshape(B, G)
    e3 = E3.reshape(B, G)
    wv = WV.reshape(B, 32)
    W1x = jnp.pad(W1, ((0, 512 - F * D), (0, 0)))
    return _tc_head(e0, e1, e2, e3, wv, W1x, b1, W2, b2, W3, b3, Wd, bd)


# speedup vs baseline: 1.8233x; 1.8233x over previous
import functools

import jax
import jax.numpy as jnp
from jax import lax
from jax.experimental import pallas as pl
from jax.experimental.pallas import tpu as pltpu
from jax.experimental.pallas import tpu_sc as plsc

B = 16384
F = 26
V = 100000
D = 16
H = 200

NC = 2
NS = 16
NW = NC * NS

G = 128
NEG = B // 16
EGPW = NEG // NW
NWG = (B * 32) // G
WGPW = NWG // NW


def _sc_gather(i0, i1, i2, i3, iw, tab2d, w_flat):
    mesh = plsc.VectorSubcoreMesh(core_axis_name="c", subcore_axis_name="s")

    @functools.partial(
        pl.kernel,
        mesh=mesh,
        compiler_params=pltpu.CompilerParams(use_tc_tiling_on_sc=False),
        out_type=[
            jax.ShapeDtypeStruct((NEG, G, D), jnp.float32),
            jax.ShapeDtypeStruct((NEG, G, D), jnp.float32),
            jax.ShapeDtypeStruct((NEG, G, D), jnp.float32),
            jax.ShapeDtypeStruct((NEG, G, D), jnp.float32),
            jax.ShapeDtypeStruct((NWG, G), jnp.float32),
        ],
        scratch_types=[
            pltpu.VMEM((4 * EGPW, G), jnp.int32),
            pltpu.VMEM((WGPW, G), jnp.int32),
            pltpu.VMEM((G, D), jnp.float32),
            pltpu.VMEM((G, D), jnp.float32),
            pltpu.VMEM((G,), jnp.float32),
            pltpu.VMEM((G,), jnp.float32),
            pltpu.VMEM((G,), jnp.float32),
            pltpu.VMEM((G,), jnp.float32),
            pltpu.SemaphoreType.DMA,
            pltpu.SemaphoreType.DMA,
            pltpu.SemaphoreType.DMA,
            pltpu.SemaphoreType.DMA,
        ],
    )
    def gather_kernel(i0_h, i1_h, i2_h, i3_h, iw_h, tab_h, w_h,
                      e0_h, e1_h, e2_h, e3_h, wv_h,
                      idxe_v, idxw_v, bufa, bufb, wba, wbb, wbc, wbd,
                      sema, semb, semc, semd):
        wid = lax.axis_index("s") * NC + lax.axis_index("c")
        eb = wid * EGPW
        wb = wid * WGPW
        for q, iq_h in enumerate((i0_h, i1_h, i2_h, i3_h)):
            pltpu.sync_copy(iq_h.at[pl.ds(eb, EGPW)],
                            idxe_v.at[pl.ds(q * EGPW, EGPW)])
        pltpu.sync_copy(iw_h.at[pl.ds(wb, WGPW)], idxw_v)

        for q, eq_h in enumerate((e0_h, e1_h, e2_h, e3_h)):
            def ebody(j, carry, q=q, eq_h=eq_h):
                k = 2 * j
                cpa = pltpu.async_copy(
                    tab_h.at[idxe_v.at[q * EGPW + k]], bufa, sema)
                cpb = pltpu.async_copy(
                    tab_h.at[idxe_v.at[q * EGPW + k + 1]], bufb, semb)
                cpa.wait()
                pltpu.sync_copy(bufa, eq_h.at[eb + k])
                cpb.wait()
                pltpu.sync_copy(bufb, eq_h.at[eb + k + 1])
                return carry
            lax.fori_loop(0, EGPW // 2, ebody, 0)

        def wbody(j, carry):
            g = 4 * j
            cps = []
            for t, (wbuf, sem) in enumerate(
                    ((wba, sema), (wbb, semb), (wbc, semc), (wbd, semd))):
                cps.append(pltpu.async_copy(
                    w_h.at[idxw_v.at[g + t]], wbuf, sem))
            for t, (wbuf, _) in enumerate(
                    ((wba, sema), (wbb, semb), (wbc, semc), (wbd, semd))):
                cps[t].wait()
                pltpu.sync_copy(wbuf, wv_h.at[wb + g + t])
            return carry
        lax.fori_loop(0, WGPW // 4, wbody, 0)

    return gather_kernel(i0, i1, i2, i3, iw, tab2d, w_flat)


KV = 11
CW = KV * 128
CPF = 781 // KV
NCHK = F * CPF
CPW = 58
CROWS = CW // 8
VMAIN = 781 * 128
MROWS = F * (VMAIN // 8)
TROWS = MROWS + F * 8


def _sc_detile(tab_fd, tail):
    mesh = plsc.VectorSubcoreMesh(core_axis_name="c", subcore_axis_name="s")

    @functools.partial(
        pl.kernel,
        mesh=mesh,
        compiler_params=pltpu.CompilerParams(use_tc_tiling_on_sc=False,
                                             needs_layout_passes=False),
        out_type=[jax.ShapeDtypeStruct((TROWS, 128), jnp.float32)],
        scratch_types=[
            pltpu.VMEM((D, CW + 1), jnp.float32),
            pltpu.VMEM((D, CW + 1), jnp.float32),
            pltpu.VMEM((CROWS, 128), jnp.float32),
            pltpu.VMEM((CROWS, 128), jnp.float32),
            pltpu.SemaphoreType.DMA,
            pltpu.SemaphoreType.DMA,
            pltpu.SemaphoreType.DMA,
            pltpu.SemaphoreType.DMA,
        ],
    )
    def detile_kernel(tab_h, tail_h, out_h, bufa, bufb, ova, ovb,
                      sia, sib, soa, sob):
        wid = lax.axis_index("s") * NC + lax.axis_index("c")
        iota16 = lax.broadcasted_iota(jnp.int32, (D,), 0)

        def chunk_c(k):
            return k * NW + wid

        def fire_in(c, buf, sem):
            f = c // CPF
            vq = c - f * CPF
            pltpu.async_copy(
                tab_h.at[pl.ds(D * f, D), pl.ds(CW * vq, CW)],
                buf.at[pl.ds(0, D), pl.ds(0, CW)], sem)

        def wait_in(buf, sem):
            pltpu.make_async_copy(
                tab_h.at[pl.ds(0, D), pl.ds(0, CW)],
                buf.at[pl.ds(0, D), pl.ds(0, CW)], sem).wait()

        def shuffle(buf, ov):
            def jbody(j, carry):
                j8 = 8 * j
                for tt in range(KV):
                    vals = []
                    for m in range(8):
                        colv = jnp.full((D,), 128 * tt + m, jnp.int32) + j8
                        vals.append(plsc.load_gather(buf, [iota16, colv]))
                    for m in range(8):
                        ov[D * tt + j, pl.ds(D * m, D)] = vals[m]
                return carry
            lax.fori_loop(0, 16, jbody, 0)

        def fire_out(c, ov, sem):
            pltpu.async_copy(ov, out_h.at[pl.ds(CROWS * c, CROWS)], sem)

        def wait_out(ov, sem):
            pltpu.make_async_copy(
                out_h.at[pl.ds(0, CROWS)], ov, sem).wait()

        @pl.when(chunk_c(0) < NCHK)
        def _():
            fire_in(chunk_c(0), bufa, sia)

        @pl.when(chunk_c(1) < NCHK)
        def _():
            fire_in(chunk_c(1), bufb, sib)

        def body(i, carry):
            k0 = 2 * i
            for par, (buf, ov, si, so) in enumerate(
                    ((bufa, ova, sia, soa), (bufb, ovb, sib, sob))):
                k = k0 + par
                c = chunk_c(k)

                @pl.when(c < NCHK)
                def _(buf=buf, ov=ov, si=si, so=so, k=k, c=c):
                    wait_in(buf, si)

                    @pl.when(k >= 2)
                    def _(ov=ov, so=so):
                        wait_out(ov, so)
                    shuffle(buf, ov)

                    @pl.when(chunk_c(k + 2) < NCHK)
                    def _(buf=buf, si=si, k=k):
                        fire_in(chunk_c(k + 2), buf, si)
                    fire_out(c, ov, so)
            return carry

        lax.fori_loop(0, CPW // 2, body, 0)

        @pl.when(chunk_c(CPW - 2) < NCHK)
        def _():
            wait_out(ova, soa)

        @pl.when(chunk_c(CPW - 1) < NCHK)
        def _():
            wait_out(ovb, sob)

        @pl.when(wid < F)
        def _():
            pltpu.sync_copy(tail_h.at[wid],
                            out_h.at[pl.ds(MROWS + 8 * wid, 8)])

    return detile_kernel(tab_fd, tail)[0]


BLK = 512


def _tc_head(e0, e1, e2, e3, wv, W1x, b1, W2, b2, W3, b3, Wd, bd):
    def body(e0_ref, e1_ref, e2_ref, e3_ref, wv_ref, W1x_ref, b1_ref,
             W2_ref, b2_ref, W3_ref, b3_ref, Wd_ref, bd_ref, out_ref):
        E0, E1, E2, E3 = e0_ref[...], e1_ref[...], e2_ref[...], e3_ref[...]
        m3 = (lax.broadcasted_iota(jnp.int32, (1, G), 1) < 2 * D).astype(
            jnp.float32)
        E3m = E3 * m3
        rj = lax.broadcasted_iota(jnp.int32, (G, D), 0)
        cd = lax.broadcasted_iota(jnp.int32, (G, D), 1)
        S = jnp.where((rj % D) == cd, 1.0, 0.0).astype(jnp.float32)
        Esum = E0 + E1 + E2 + E3m
        sum_d = jnp.dot(Esum, S, preferred_element_type=jnp.float32)
        sos = jnp.sum(sum_d * sum_d, axis=1, keepdims=True)
        ssq = jnp.sum(E0 * E0 + E1 * E1 + E2 * E2 + E3m * E3m,
                      axis=1, keepdims=True)
        second = 0.5 * (sos - ssq)
        mw = (lax.broadcasted_iota(jnp.int32, (1, 32), 1) < F).astype(
            jnp.float32)
        first = jnp.sum(wv_ref[...] * mw, axis=1, keepdims=True)
        W1x = W1x_ref[...]
        h = (jnp.dot(E0, W1x[0:128], preferred_element_type=jnp.float32)
             + jnp.dot(E1, W1x[128:256], preferred_element_type=jnp.float32)
             + jnp.dot(E2, W1x[256:384], preferred_element_type=jnp.float32)
             + jnp.dot(E3, W1x[384:512], preferred_element_type=jnp.float32))
        h = jnp.maximum(h + b1_ref[...], 0.0)
        h = jnp.maximum(
            jnp.dot(h, W2_ref[...], preferred_element_type=jnp.float32)
            + b2_ref[...], 0.0)
        h = jnp.maximum(
            jnp.dot(h, W3_ref[...], preferred_element_type=jnp.float32)
            + b3_ref[...], 0.0)
        deep = jnp.dot(h, Wd_ref[...], preferred_element_type=jnp.float32) \
            + bd_ref[...]
        out_ref[...] = jax.nn.sigmoid(first + second + deep)

    return pl.pallas_call(
        body,
        grid=(B // BLK,),
        in_specs=[
            pl.BlockSpec((BLK, G), lambda i: (i, 0)),
            pl.BlockSpec((BLK, G), lambda i: (i, 0)),
            pl.BlockSpec((BLK, G), lambda i: (i, 0)),
            pl.BlockSpec((BLK, G), lambda i: (i, 0)),
            pl.BlockSpec((BLK, 32), lambda i: (i, 0)),
            pl.BlockSpec((512, H), lambda i: (0, 0)),
            pl.BlockSpec((1, H), lambda i: (0, 0)),
            pl.BlockSpec((H, H), lambda i: (0, 0)),
            pl.BlockSpec((1, H), lambda i: (0, 0)),
            pl.BlockSpec((H, H), lambda i: (0, 0)),
            pl.BlockSpec((1, H), lambda i: (0, 0)),
            pl.BlockSpec((H, 1), lambda i: (0, 0)),
            pl.BlockSpec((1, 1), lambda i: (0, 0)),
        ],
        out_specs=pl.BlockSpec((BLK, 1), lambda i: (i, 0)),
        out_shape=jax.ShapeDtypeStruct((B, 1), jnp.float32),
    )(e0, e1, e2, e3, wv, W1x, b1.reshape(1, H), W2, b2.reshape(1, H),
      W3, b3.reshape(1, H), Wd, bd.reshape(1, 1))


def kernel(indices, tables, w_fm, W1, b1, W2, b2, W3, b3, Wd, bd):
    frow = jnp.arange(F, dtype=jnp.int32)[None, :]
    flat = indices + frow * V
    w32 = jnp.concatenate([flat, flat[:, :32 - F]], axis=1)
    iw = w32.reshape(NWG, G)
    v = indices
    vq = v // CW
    vloc = v - vq * CW
    main_row = 8 * (CROWS * (frow * CPF + vq) + vloc // 8) + (v % 8)
    tail_row = 8 * (MROWS + 8 * frow + (v - (V - 64)) // 8) + (v % 8)
    srow = jnp.where(v < VMAIN, main_row, tail_row)
    s32 = jnp.concatenate([srow, srow[:, :32 - F]], axis=1)
    iqs = [s32[:, 8 * q:8 * q + 8].reshape(NEG, G) for q in range(4)]
    tab_fd = tables.transpose(0, 2, 1).reshape(F * D, V)
    tail = tables[:, V - 64:, :].reshape(F, 8, 8 * D)
    tab2d = _sc_detile(tab_fd, tail).reshape(TROWS * 8, D)
    w_flat = w_fm.reshape(F * V)
    E0, E1, E2, E3, WV = _sc_gather(iqs[0], iqs[1], iqs[2], iqs[3], iw,
                                    tab2d, w_flat)
    e0 = E0.reshape(B, G)
    e1 = E1.reshape(B, G)
    e2 = E2.reshape(B, G)
    e3 = E3.reshape(B, G)
    wv = WV.reshape(B, 32)
    W1x = jnp.pad(W1, ((0, 512 - F * D), (0, 0)))
    return _tc_head(e0, e1, e2, e3, wv, W1x, b1, W2, b2, W3, b3, Wd, bd)


# speedup vs baseline: 1.9421x; 1.0651x over previous
import functools

import jax
import jax.numpy as jnp
from jax import lax
from jax.experimental import pallas as pl
from jax.experimental.pallas import tpu as pltpu
from jax.experimental.pallas import tpu_sc as plsc

B = 16384
F = 26
V = 100000
D = 16
H = 200

NC = 2
NS = 16
NW = NC * NS

G = 128
NEG = B // 16
EGPW = NEG // NW
NWG = (B * 32) // G
WGPW = NWG // NW


def _sc_gather(i0, i1, i2, i3, iw, tab2d, w_flat):
    mesh = plsc.VectorSubcoreMesh(core_axis_name="c", subcore_axis_name="s")

    @functools.partial(
        pl.kernel,
        mesh=mesh,
        compiler_params=pltpu.CompilerParams(use_tc_tiling_on_sc=False),
        out_type=[
            jax.ShapeDtypeStruct((NEG, G, D), jnp.float32),
            jax.ShapeDtypeStruct((NEG, G, D), jnp.float32),
            jax.ShapeDtypeStruct((NEG, G, D), jnp.float32),
            jax.ShapeDtypeStruct((NEG, G, D), jnp.float32),
            jax.ShapeDtypeStruct((NWG, G), jnp.float32),
        ],
        scratch_types=[
            pltpu.VMEM((4 * EGPW, G), jnp.int32),
            pltpu.VMEM((WGPW, G), jnp.int32),
            pltpu.VMEM((G, D), jnp.float32),
            pltpu.VMEM((G, D), jnp.float32),
            pltpu.VMEM((G, D), jnp.float32),
            pltpu.VMEM((G, D), jnp.float32),
            pltpu.VMEM((4, G), jnp.float32),
            pltpu.VMEM((4, G), jnp.float32),
            pltpu.SemaphoreType.DMA,
            pltpu.SemaphoreType.DMA,
            pltpu.SemaphoreType.DMA,
            pltpu.SemaphoreType.DMA,
            pltpu.SemaphoreType.DMA,
            pltpu.SemaphoreType.DMA,
            pltpu.SemaphoreType.DMA,
            pltpu.SemaphoreType.DMA,
        ],
    )
    def gather_kernel(i0_h, i1_h, i2_h, i3_h, iw_h, tab_h, w_h,
                      e0_h, e1_h, e2_h, e3_h, wv_h,
                      idxe_v, idxw_v, bufa, bufb, bufc, bufd, wva, wvb,
                      sema, semb, semc, semd, soa, sob, soc, sod):
        wid = lax.axis_index("s") * NC + lax.axis_index("c")
        eb = wid * EGPW
        wb = wid * WGPW
        for q, iq_h in enumerate((i0_h, i1_h, i2_h, i3_h)):
            pltpu.sync_copy(iq_h.at[pl.ds(eb, EGPW)],
                            idxe_v.at[pl.ds(q * EGPW, EGPW)])
        pltpu.sync_copy(iw_h.at[pl.ds(wb, WGPW)], idxw_v)

        ebufs = ((bufa, sema, soa), (bufb, semb, sob),
                 (bufc, semc, soc), (bufd, semd, sod))
        for q, eq_h in enumerate((e0_h, e1_h, e2_h, e3_h)):
            def ebody(j, carry, q=q, eq_h=eq_h):
                k = 4 * j

                @pl.when(j > 0)
                def _(eq_h=eq_h):
                    for buf, _, so in ebufs:
                        pltpu.make_async_copy(buf, eq_h.at[eb], so).wait()
                cps = [pltpu.async_copy(
                    tab_h.at[idxe_v.at[q * EGPW + k + t]], buf, sem)
                    for t, (buf, sem, _) in enumerate(ebufs)]
                for t, (buf, _, so) in enumerate(ebufs):
                    cps[t].wait()
                    pltpu.async_copy(buf, eq_h.at[eb + k + t], so)
                return carry
            lax.fori_loop(0, EGPW // 4, ebody, 0)
            for buf, _, so in ebufs:
                pltpu.make_async_copy(buf, eq_h.at[eb], so).wait()

        def wbody(j, carry):
            g = 8 * j

            @pl.when(j > 0)
            def _():
                pltpu.make_async_copy(wva, wv_h.at[pl.ds(wb, 4)], soa).wait()
                pltpu.make_async_copy(wvb, wv_h.at[pl.ds(wb, 4)], sob).wait()
            cps = []
            for t in range(4):
                cps.append(pltpu.async_copy(
                    w_h.at[idxw_v.at[g + t]], wva.at[t], sema))
            for t in range(4):
                cps.append(pltpu.async_copy(
                    w_h.at[idxw_v.at[g + 4 + t]], wvb.at[t], semb))
            for cp in cps[:4]:
                cp.wait()
            pltpu.async_copy(wva, wv_h.at[pl.ds(wb + g, 4)], soa)
            for cp in cps[4:]:
                cp.wait()
            pltpu.async_copy(wvb, wv_h.at[pl.ds(wb + g + 4, 4)], sob)
            return carry
        lax.fori_loop(0, WGPW // 8, wbody, 0)
        pltpu.make_async_copy(wva, wv_h.at[pl.ds(wb, 4)], soa).wait()
        pltpu.make_async_copy(wvb, wv_h.at[pl.ds(wb, 4)], sob).wait()

    return gather_kernel(i0, i1, i2, i3, iw, tab2d, w_flat)


KV = 11
CW = KV * 128
CPF = 781 // KV
NCHK = F * CPF
CPW = 58
CROWS = CW // 8
VMAIN = 781 * 128
MROWS = F * (VMAIN // 8)
TROWS = MROWS + F * 8


def _sc_detile(tab_fd, tail):
    mesh = plsc.VectorSubcoreMesh(core_axis_name="c", subcore_axis_name="s")

    @functools.partial(
        pl.kernel,
        mesh=mesh,
        compiler_params=pltpu.CompilerParams(use_tc_tiling_on_sc=False,
                                             needs_layout_passes=False),
        out_type=[jax.ShapeDtypeStruct((TROWS, 128), jnp.float32)],
        scratch_types=[
            pltpu.VMEM((D, CW + 1), jnp.float32),
            pltpu.VMEM((D, CW + 1), jnp.float32),
            pltpu.VMEM((CROWS, 128), jnp.float32),
            pltpu.VMEM((CROWS, 128), jnp.float32),
            pltpu.SemaphoreType.DMA,
            pltpu.SemaphoreType.DMA,
            pltpu.SemaphoreType.DMA,
            pltpu.SemaphoreType.DMA,
        ],
    )
    def detile_kernel(tab_h, tail_h, out_h, bufa, bufb, ova, ovb,
                      sia, sib, soa, sob):
        wid = lax.axis_index("s") * NC + lax.axis_index("c")
        iota16 = lax.broadcasted_iota(jnp.int32, (D,), 0)

        def chunk_c(k):
            return k * NW + wid

        def fire_in(c, buf, sem):
            f = c // CPF
            vq = c - f * CPF
            pltpu.async_copy(
                tab_h.at[pl.ds(D * f, D), pl.ds(CW * vq, CW)],
                buf.at[pl.ds(0, D), pl.ds(0, CW)], sem)

        def wait_in(buf, sem):
            pltpu.make_async_copy(
                tab_h.at[pl.ds(0, D), pl.ds(0, CW)],
                buf.at[pl.ds(0, D), pl.ds(0, CW)], sem).wait()

        def shuffle(buf, ov):
            def jbody(j, carry):
                j8 = 8 * j
                for tt in range(KV):
                    vals = []
                    for m in range(8):
                        colv = jnp.full((D,), 128 * tt + m, jnp.int32) + j8
                        vals.append(plsc.load_gather(buf, [iota16, colv]))
                    for m in range(8):
                        ov[D * tt + j, pl.ds(D * m, D)] = vals[m]
                return carry
            lax.fori_loop(0, 16, jbody, 0)

        def fire_out(c, ov, sem):
            pltpu.async_copy(ov, out_h.at[pl.ds(CROWS * c, CROWS)], sem)

        def wait_out(ov, sem):
            pltpu.make_async_copy(
                out_h.at[pl.ds(0, CROWS)], ov, sem).wait()

        @pl.when(chunk_c(0) < NCHK)
        def _():
            fire_in(chunk_c(0), bufa, sia)

        @pl.when(chunk_c(1) < NCHK)
        def _():
            fire_in(chunk_c(1), bufb, sib)

        def body(i, carry):
            k0 = 2 * i
            for par, (buf, ov, si, so) in enumerate(
                    ((bufa, ova, sia, soa), (bufb, ovb, sib, sob))):
                k = k0 + par
                c = chunk_c(k)

                @pl.when(c < NCHK)
                def _(buf=buf, ov=ov, si=si, so=so, k=k, c=c):
                    wait_in(buf, si)

                    @pl.when(k >= 2)
                    def _(ov=ov, so=so):
                        wait_out(ov, so)
                    shuffle(buf, ov)

                    @pl.when(chunk_c(k + 2) < NCHK)
                    def _(buf=buf, si=si, k=k):
                        fire_in(chunk_c(k + 2), buf, si)
                    fire_out(c, ov, so)
            return carry

        lax.fori_loop(0, CPW // 2, body, 0)

        @pl.when(chunk_c(CPW - 2) < NCHK)
        def _():
            wait_out(ova, soa)

        @pl.when(chunk_c(CPW - 1) < NCHK)
        def _():
            wait_out(ovb, sob)

        @pl.when(wid < F)
        def _():
            pltpu.sync_copy(tail_h.at[wid],
                            out_h.at[pl.ds(MROWS + 8 * wid, 8)])

    return detile_kernel(tab_fd, tail)[0]


BLK = 512


def _tc_head(e0, e1, e2, e3, wv, W1x, b1, W2, b2, W3, b3, Wd, bd):
    def body(e0_ref, e1_ref, e2_ref, e3_ref, wv_ref, W1x_ref, b1_ref,
             W2_ref, b2_ref, W3_ref, b3_ref, Wd_ref, bd_ref, out_ref):
        E0, E1, E2, E3 = e0_ref[...], e1_ref[...], e2_ref[...], e3_ref[...]
        m3 = (lax.broadcasted_iota(jnp.int32, (1, G), 1) < 2 * D).astype(
            jnp.float32)
        E3m = E3 * m3
        rj = lax.broadcasted_iota(jnp.int32, (G, D), 0)
        cd = lax.broadcasted_iota(jnp.int32, (G, D), 1)
        S = jnp.where((rj % D) == cd, 1.0, 0.0).astype(jnp.float32)
        Esum = E0 + E1 + E2 + E3m
        sum_d = jnp.dot(Esum, S, preferred_element_type=jnp.float32)
        sos = jnp.sum(sum_d * sum_d, axis=1, keepdims=True)
        ssq = jnp.sum(E0 * E0 + E1 * E1 + E2 * E2 + E3m * E3m,
                      axis=1, keepdims=True)
        second = 0.5 * (sos - ssq)
        mw = (lax.broadcasted_iota(jnp.int32, (1, 32), 1) < F).astype(
            jnp.float32)
        first = jnp.sum(wv_ref[...] * mw, axis=1, keepdims=True)
        W1x = W1x_ref[...]
        h = (jnp.dot(E0, W1x[0:128], preferred_element_type=jnp.float32)
             + jnp.dot(E1, W1x[128:256], preferred_element_type=jnp.float32)
             + jnp.dot(E2, W1x[256:384], preferred_element_type=jnp.float32)
             + jnp.dot(E3, W1x[384:512], preferred_element_type=jnp.float32))
        h = jnp.maximum(h + b1_ref[...], 0.0)
        h = jnp.maximum(
            jnp.dot(h, W2_ref[...], preferred_element_type=jnp.float32)
            + b2_ref[...], 0.0)
        h = jnp.maximum(
            jnp.dot(h, W3_ref[...], preferred_element_type=jnp.float32)
            + b3_ref[...], 0.0)
        deep = jnp.dot(h, Wd_ref[...], preferred_element_type=jnp.float32) \
            + bd_ref[...]
        out_ref[...] = jax.nn.sigmoid(first + second + deep)

    return pl.pallas_call(
        body,
        grid=(B // BLK,),
        in_specs=[
            pl.BlockSpec((BLK, G), lambda i: (i, 0)),
            pl.BlockSpec((BLK, G), lambda i: (i, 0)),
            pl.BlockSpec((BLK, G), lambda i: (i, 0)),
            pl.BlockSpec((BLK, G), lambda i: (i, 0)),
            pl.BlockSpec((BLK, 32), lambda i: (i, 0)),
            pl.BlockSpec((512, H), lambda i: (0, 0)),
            pl.BlockSpec((1, H), lambda i: (0, 0)),
            pl.BlockSpec((H, H), lambda i: (0, 0)),
            pl.BlockSpec((1, H), lambda i: (0, 0)),
            pl.BlockSpec((H, H), lambda i: (0, 0)),
            pl.BlockSpec((1, H), lambda i: (0, 0)),
            pl.BlockSpec((H, 1), lambda i: (0, 0)),
            pl.BlockSpec((1, 1), lambda i: (0, 0)),
        ],
        out_specs=pl.BlockSpec((BLK, 1), lambda i: (i, 0)),
        out_shape=jax.ShapeDtypeStruct((B, 1), jnp.float32),
    )(e0, e1, e2, e3, wv, W1x, b1.reshape(1, H), W2, b2.reshape(1, H),
      W3, b3.reshape(1, H), Wd, bd.reshape(1, 1))


def kernel(indices, tables, w_fm, W1, b1, W2, b2, W3, b3, Wd, bd):
    frow = jnp.arange(F, dtype=jnp.int32)[None, :]
    flat = indices + frow * V
    w32 = jnp.concatenate([flat, flat[:, :32 - F]], axis=1)
    iw = w32.reshape(NWG, G)
    v = indices
    vq = v // CW
    vloc = v - vq * CW
    main_row = 8 * (CROWS * (frow * CPF + vq) + vloc // 8) + (v % 8)
    tail_row = 8 * (MROWS + 8 * frow + (v - (V - 64)) // 8) + (v % 8)
    srow = jnp.where(v < VMAIN, main_row, tail_row)
    s32 = jnp.concatenate([srow, srow[:, :32 - F]], axis=1)
    iqs = [s32[:, 8 * q:8 * q + 8].reshape(NEG, G) for q in range(4)]
    tab_fd = tables.transpose(0, 2, 1).reshape(F * D, V)
    tail = tables[:, V - 64:, :].reshape(F, 8, 8 * D)
    tab2d = _sc_detile(tab_fd, tail).reshape(TROWS * 8, D)
    w_flat = w_fm.reshape(F * V)
    E0, E1, E2, E3, WV = _sc_gather(iqs[0], iqs[1], iqs[2], iqs[3], iw,
                                    tab2d, w_flat)
    e0 = E0.reshape(B, G)
    e1 = E1.reshape(B, G)
    e2 = E2.reshape(B, G)
    e3 = E3.reshape(B, G)
    wv = WV.reshape(B, 32)
    W1x = jnp.pad(W1, ((0, 512 - F * D), (0, 0)))
    return _tc_head(e0, e1, e2, e3, wv, W1x, b1, W2, b2, W3, b3, Wd, bd)
